# baseline (speedup 1.0000x reference)
import jax
import jax.numpy as jnp
from jax.experimental import pallas as pl


_BLOCK_N = 2048


def _scoring_kernel(x_ref, w_ref, b_ref, o_ref):
    acc = jax.lax.dot_general(
        x_ref[...],
        w_ref[...],
        (((1,), (1,)), ((), ())),
        preferred_element_type=jnp.float32,
    )
    o_ref[...] = acc + b_ref[...]


def kernel(x, y, freeze_flag, W, b):
    del y, freeze_flag
    B, D = x.shape
    N = W.shape[0]
    b2 = b.reshape(1, N)
    out = pl.pallas_call(
        _scoring_kernel,
        grid=(pl.cdiv(N, _BLOCK_N),),
        in_specs=[
            pl.BlockSpec((B, D), lambda i: (0, 0)),
            pl.BlockSpec((_BLOCK_N, D), lambda i: (i, 0)),
            pl.BlockSpec((1, _BLOCK_N), lambda i: (0, i)),
        ],
        out_specs=pl.BlockSpec((B, _BLOCK_N), lambda i: (0, i)),
        out_shape=jax.ShapeDtypeStruct((B, N), jnp.float32),
    )(x, W, b2)
    return out


# speedup vs baseline: 1.1927x; 1.1927x over previous
import jax
import jax.numpy as jnp
from jax.experimental import pallas as pl
from jax.experimental.pallas import tpu as pltpu


_BLOCK_N = 4096


def _scoring_kernel(x_ref, w_ref, b_ref, o_ref):
    acc = jax.lax.dot_general(
        x_ref[...],
        w_ref[...],
        (((1,), (1,)), ((), ())),
        preferred_element_type=jnp.float32,
    )
    o_ref[...] = acc + b_ref[...]


def kernel(x, y, freeze_flag, W, b):
    del y, freeze_flag
    B, D = x.shape
    N = W.shape[0]
    b2 = b.reshape(1, N)
    out = pl.pallas_call(
        _scoring_kernel,
        grid=(pl.cdiv(N, _BLOCK_N),),
        in_specs=[
            pl.BlockSpec((B, D), lambda i: (0, 0)),
            pl.BlockSpec((_BLOCK_N, D), lambda i: (i, 0)),
            pl.BlockSpec((1, _BLOCK_N), lambda i: (0, i)),
        ],
        out_specs=pl.BlockSpec((B, _BLOCK_N), lambda i: (0, i)),
        out_shape=jax.ShapeDtypeStruct((B, N), jnp.float32),
        compiler_params=pltpu.CompilerParams(
            dimension_semantics=("parallel",),
        ),
    )(x, W, b2)
    return out


# speedup vs baseline: 1.2707x; 1.0654x over previous
import jax
import jax.numpy as jnp
from jax.experimental import pallas as pl
from jax.experimental.pallas import tpu as pltpu


_BLOCK_N = 8192


def _scoring_kernel(x_ref, w_ref, b_ref, o_ref):
    acc = jax.lax.dot_general(
        x_ref[...],
        w_ref[...],
        (((1,), (1,)), ((), ())),
        preferred_element_type=jnp.float32,
    )
    o_ref[...] = acc + b_ref[...]


def kernel(x, y, freeze_flag, W, b):
    del y, freeze_flag
    B, D = x.shape
    N = W.shape[0]
    b2 = b.reshape(1, N)
    out = pl.pallas_call(
        _scoring_kernel,
        grid=(pl.cdiv(N, _BLOCK_N),),
        in_specs=[
            pl.BlockSpec((B, D), lambda i: (0, 0)),
            pl.BlockSpec((_BLOCK_N, D), lambda i: (i, 0)),
            pl.BlockSpec((1, _BLOCK_N), lambda i: (0, i)),
        ],
        out_specs=pl.BlockSpec((B, _BLOCK_N), lambda i: (0, i)),
        out_shape=jax.ShapeDtypeStruct((B, N), jnp.float32),
        compiler_params=pltpu.CompilerParams(
            dimension_semantics=("parallel",),
        ),
    )(x, W, b2)
    return out
